# baseline (speedup 1.0000x reference)
import functools

import jax
import jax.numpy as jnp
from jax import lax
from jax.experimental import pallas as pl
from jax.experimental.pallas import tpu as pltpu
from jax.experimental.pallas import tpu_sc as plsc

B = 4096
HIST = 50
N_CAT = 26
N_NUM = 13
NC, NS = 2, 16
NW = NC * NS
RPW = B // NW
LANES = 16
CHUNKS = RPW // LANES
SEQN = HIST * RPW
CATN = N_CAT * RPW
TOTN = 2 * SEQN + CATN

CAT_V = 100000
SEQ_V = 1000000
OFF_SEQ1 = SEQ_V
OFF_CAT = 2 * SEQ_V
TOT_V = 2 * SEQ_V + N_CAT * CAT_V
TAB_PAD = 832


def _sc_body(idx_hbm, num_hbm, wb_hbm, tab_hbm, out_hbm,
             idx_v, val_v, num_v, wb_v, res_v, sem):
    wid = lax.axis_index("s") * NC + lax.axis_index("c")
    base = wid * RPW

    pltpu.sync_copy(idx_hbm.at[wid], idx_v)
    pltpu.sync_copy(num_hbm.at[:, pl.ds(base, RPW)], num_v)
    pltpu.sync_copy(wb_hbm, wb_v)

    pltpu.async_copy(tab_hbm.at[idx_v], val_v, sem).wait()

    lanes = jax.lax.iota(jnp.int32, LANES)
    for c in range(CHUNKS):
        sl = pl.ds(c * LANES, LANES)
        acc = wb_v[N_NUM, :]
        for i in range(N_NUM):
            acc = acc + num_v[i, sl] * wb_v[i, :]

        bv = (c * LANES + lanes) * HIST

        def hbody(h, a):
            return (a + plsc.load_gather(val_v, [bv + h])
                    + plsc.load_gather(val_v, [bv + (SEQN + h)]))

        acc = lax.fori_loop(0, HIST, hbody, acc)

        def tbody(t, a):
            return a + val_v[pl.ds(2 * SEQN + t * RPW + c * LANES, LANES)]

        acc = lax.fori_loop(0, N_CAT, tbody, acc)
        res_v[sl] = acc

    pltpu.sync_copy(res_v, out_hbm.at[pl.ds(base, RPW)])


@jax.jit
def _run(idx_w, num_all, wb, tab):
    mesh = plsc.VectorSubcoreMesh(core_axis_name="c", subcore_axis_name="s")
    fn = functools.partial(
        pl.kernel,
        mesh=mesh,
        compiler_params=pltpu.CompilerParams(needs_layout_passes=False,
                                             use_tc_tiling_on_sc=False),
        out_type=jax.ShapeDtypeStruct((B,), jnp.float32),
        scratch_types=[
            pltpu.VMEM((TOTN,), jnp.int32),
            pltpu.VMEM((TOTN,), jnp.float32),
            pltpu.VMEM((N_NUM, RPW), jnp.float32),
            pltpu.VMEM((N_NUM + 1, LANES), jnp.float32),
            pltpu.VMEM((RPW,), jnp.float32),
            pltpu.SemaphoreType.DMA,
        ],
    )(_sc_body)
    return fn(idx_w, num_all, wb, tab)


def kernel(seq_0, seq_0_table, seq_1, seq_1_table, cat_0, cat_0_table, cat_1, cat_1_table, cat_2, cat_2_table, cat_3, cat_3_table, cat_4, cat_4_table, cat_5, cat_5_table, cat_6, cat_6_table, cat_7, cat_7_table, cat_8, cat_8_table, cat_9, cat_9_table, cat_10, cat_10_table, cat_11, cat_11_table, cat_12, cat_12_table, cat_13, cat_13_table, cat_14, cat_14_table, cat_15, cat_15_table, cat_16, cat_16_table, cat_17, cat_17_table, cat_18, cat_18_table, cat_19, cat_19_table, cat_20, cat_20_table, cat_21, cat_21_table, cat_22, cat_22_table, cat_23, cat_23_table, cat_24, cat_24_table, cat_25, cat_25_table, num_0, num_1, num_2, num_3, num_4, num_5, num_6, num_7, num_8, num_9, num_10, num_11, num_12, dense_W, dense_b, bn_gamma, bn_beta, bn_mean, bn_var):
    cats = [cat_0, cat_1, cat_2, cat_3, cat_4, cat_5, cat_6, cat_7, cat_8,
            cat_9, cat_10, cat_11, cat_12, cat_13, cat_14, cat_15, cat_16,
            cat_17, cat_18, cat_19, cat_20, cat_21, cat_22, cat_23, cat_24,
            cat_25]
    cat_tables = [cat_0_table, cat_1_table, cat_2_table, cat_3_table,
                  cat_4_table, cat_5_table, cat_6_table, cat_7_table,
                  cat_8_table, cat_9_table, cat_10_table, cat_11_table,
                  cat_12_table, cat_13_table, cat_14_table, cat_15_table,
                  cat_16_table, cat_17_table, cat_18_table, cat_19_table,
                  cat_20_table, cat_21_table, cat_22_table, cat_23_table,
                  cat_24_table, cat_25_table]
    nums = [num_0, num_1, num_2, num_3, num_4, num_5, num_6, num_7, num_8,
            num_9, num_10, num_11, num_12]

    tab = jnp.concatenate(
        [seq_0_table, seq_1_table] + cat_tables
        + [jnp.zeros((TAB_PAD, 1), jnp.float32)], axis=0).reshape(-1)

    seq0_i = seq_0.astype(jnp.int32).reshape(NW, SEQN)
    seq1_i = (seq_1.astype(jnp.int32) + OFF_SEQ1).reshape(NW, SEQN)
    cat_off = OFF_CAT + CAT_V * jnp.arange(N_CAT, dtype=jnp.int32)
    cat_i = jnp.concatenate(
        [c.astype(jnp.int32).reshape(1, B) for c in cats],
        axis=0) + cat_off[:, None]
    cat_w = (cat_i.reshape(N_CAT, NW, RPW)
             .transpose(1, 0, 2).reshape(NW, CATN))
    idx_w = jnp.concatenate([seq0_i, seq1_i, cat_w], axis=1)

    num_all = jnp.stack(nums, axis=0).astype(jnp.float32)

    inv = bn_gamma / jnp.sqrt(bn_var + 1e-3)
    wfold = dense_W[:, 0] * inv
    bfold = dense_b[0] + jnp.sum((bn_beta - bn_mean * inv) * dense_W[:, 0])
    wb = jnp.broadcast_to(
        jnp.concatenate([wfold, bfold[None]]).astype(jnp.float32)[:, None],
        (N_NUM + 1, LANES))

    out = _run(idx_w, num_all, wb, tab)
    return out[:, None]


# speedup vs baseline: 1.3923x; 1.3923x over previous
import functools

import jax
import jax.numpy as jnp
from jax import lax
from jax.experimental import pallas as pl
from jax.experimental.pallas import tpu as pltpu
from jax.experimental.pallas import tpu_sc as plsc

B = 4096
HIST = 50
N_CAT = 26
N_NUM = 13
NC, NS = 2, 16
NW = NC * NS
RPW = B // NW
LANES = 16
CHUNKS = RPW // LANES
SEQN = HIST * RPW
CATN = N_CAT * RPW
TOTN = 2 * SEQN + CATN

PAD_CAT = 100352
PAD_SEQ = 1000448
OFF_SEQ1 = PAD_SEQ
OFF_CAT = 2 * PAD_SEQ
TOT_VP = 2 * PAD_SEQ + N_CAT * PAD_CAT


def _sc_body(idx_hbm, num_hbm, wb_hbm, tab_hbm, out_hbm,
             idx_v, val_v, num_v, wb_v, res_v, sem):
    wid = lax.axis_index("s") * NC + lax.axis_index("c")
    base = wid * RPW

    pltpu.sync_copy(idx_hbm.at[wid], idx_v)
    pltpu.sync_copy(num_hbm.at[:, pl.ds(base, RPW)], num_v)
    pltpu.sync_copy(wb_hbm, wb_v)

    pltpu.async_copy(tab_hbm.at[idx_v], val_v, sem).wait()

    lanes = jax.lax.iota(jnp.int32, LANES)
    for c in range(CHUNKS):
        sl = pl.ds(c * LANES, LANES)
        acc = wb_v[N_NUM, :]
        for i in range(N_NUM):
            acc = acc + num_v[i, sl] * wb_v[i, :]

        bv = (c * LANES + lanes) * HIST

        def hbody(h, a):
            return (a + plsc.load_gather(val_v, [bv + h])
                    + plsc.load_gather(val_v, [bv + (SEQN + h)]))

        acc = lax.fori_loop(0, HIST, hbody, acc)

        def tbody(t, a):
            return a + val_v[pl.ds(2 * SEQN + t * RPW + c * LANES, LANES)]

        acc = lax.fori_loop(0, N_CAT, tbody, acc)
        res_v[sl] = acc

    pltpu.sync_copy(res_v, out_hbm.at[pl.ds(base, RPW)])


@jax.jit
def _run(idx_w, num_all, wb, tab):
    mesh = plsc.VectorSubcoreMesh(core_axis_name="c", subcore_axis_name="s")
    fn = functools.partial(
        pl.kernel,
        mesh=mesh,
        compiler_params=pltpu.CompilerParams(needs_layout_passes=False,
                                             use_tc_tiling_on_sc=False),
        out_type=jax.ShapeDtypeStruct((B,), jnp.float32),
        scratch_types=[
            pltpu.VMEM((TOTN,), jnp.int32),
            pltpu.VMEM((TOTN,), jnp.float32),
            pltpu.VMEM((N_NUM, RPW), jnp.float32),
            pltpu.VMEM((N_NUM + 1, LANES), jnp.float32),
            pltpu.VMEM((RPW,), jnp.float32),
            pltpu.SemaphoreType.DMA,
        ],
    )(_sc_body)
    return fn(idx_w, num_all, wb, tab)


def kernel(seq_0, seq_0_table, seq_1, seq_1_table, cat_0, cat_0_table, cat_1, cat_1_table, cat_2, cat_2_table, cat_3, cat_3_table, cat_4, cat_4_table, cat_5, cat_5_table, cat_6, cat_6_table, cat_7, cat_7_table, cat_8, cat_8_table, cat_9, cat_9_table, cat_10, cat_10_table, cat_11, cat_11_table, cat_12, cat_12_table, cat_13, cat_13_table, cat_14, cat_14_table, cat_15, cat_15_table, cat_16, cat_16_table, cat_17, cat_17_table, cat_18, cat_18_table, cat_19, cat_19_table, cat_20, cat_20_table, cat_21, cat_21_table, cat_22, cat_22_table, cat_23, cat_23_table, cat_24, cat_24_table, cat_25, cat_25_table, num_0, num_1, num_2, num_3, num_4, num_5, num_6, num_7, num_8, num_9, num_10, num_11, num_12, dense_W, dense_b, bn_gamma, bn_beta, bn_mean, bn_var):
    cats = [cat_0, cat_1, cat_2, cat_3, cat_4, cat_5, cat_6, cat_7, cat_8,
            cat_9, cat_10, cat_11, cat_12, cat_13, cat_14, cat_15, cat_16,
            cat_17, cat_18, cat_19, cat_20, cat_21, cat_22, cat_23, cat_24,
            cat_25]
    cat_tables = [cat_0_table, cat_1_table, cat_2_table, cat_3_table,
                  cat_4_table, cat_5_table, cat_6_table, cat_7_table,
                  cat_8_table, cat_9_table, cat_10_table, cat_11_table,
                  cat_12_table, cat_13_table, cat_14_table, cat_15_table,
                  cat_16_table, cat_17_table, cat_18_table, cat_19_table,
                  cat_20_table, cat_21_table, cat_22_table, cat_23_table,
                  cat_24_table, cat_25_table]
    nums = [num_0, num_1, num_2, num_3, num_4, num_5, num_6, num_7, num_8,
            num_9, num_10, num_11, num_12]

    base = jnp.zeros((TOT_VP, 1), jnp.float32)
    base = lax.dynamic_update_slice(base, seq_0_table, (0, 0))
    base = lax.dynamic_update_slice(base, seq_1_table, (OFF_SEQ1, 0))
    for t, tb in enumerate(cat_tables):
        base = lax.dynamic_update_slice(base, tb, (OFF_CAT + t * PAD_CAT, 0))
    tab = base.reshape(-1)

    seq0_i = seq_0.astype(jnp.int32).reshape(NW, SEQN)
    seq1_i = (seq_1.astype(jnp.int32) + OFF_SEQ1).reshape(NW, SEQN)
    cat_off = OFF_CAT + PAD_CAT * jnp.arange(N_CAT, dtype=jnp.int32)
    cat_i = jnp.concatenate(
        [c.astype(jnp.int32).reshape(1, B) for c in cats],
        axis=0) + cat_off[:, None]
    cat_w = (cat_i.reshape(N_CAT, NW, RPW)
             .transpose(1, 0, 2).reshape(NW, CATN))
    idx_w = jnp.concatenate([seq0_i, seq1_i, cat_w], axis=1)

    num_all = jnp.stack(nums, axis=0).astype(jnp.float32)

    inv = bn_gamma / jnp.sqrt(bn_var + 1e-3)
    wfold = dense_W[:, 0] * inv
    bfold = dense_b[0] + jnp.sum((bn_beta - bn_mean * inv) * dense_W[:, 0])
    wb = jnp.broadcast_to(
        jnp.concatenate([wfold, bfold[None]]).astype(jnp.float32)[:, None],
        (N_NUM + 1, LANES))

    out = _run(idx_w, num_all, wb, tab)
    return out[:, None]


# speedup vs baseline: 3.4804x; 2.4997x over previous
import functools

import jax
import jax.numpy as jnp
from jax import lax
from jax.experimental import pallas as pl
from jax.experimental.pallas import tpu as pltpu
from jax.experimental.pallas import tpu_sc as plsc

B = 4096
HIST = 50
N_CAT = 26
N_NUM = 13
NC, NS = 2, 16
NW = NC * NS
RPW = B // NW
LANES = 16
CHUNKS = RPW // LANES
SEQN = HIST * RPW
CATN = N_CAT * RPW
TOTN = 2 * SEQN + CATN

PAD_CAT = 100352
PAD_SEQ = 1000448
OFF_SEQ1 = PAD_SEQ
OFF_CAT = 2 * PAD_SEQ
TOT_VP = 2 * PAD_SEQ + N_CAT * PAD_CAT


def _sc_body(idx_hbm, num_hbm, wb_hbm, tab_hbm, out_hbm,
             idx_v, val_v, num_v, wb_v, res_v, sem):
    wid = lax.axis_index("s") * NC + lax.axis_index("c")
    base = wid * RPW

    pltpu.sync_copy(idx_hbm.at[wid], idx_v)
    pltpu.sync_copy(num_hbm.at[:, pl.ds(base, RPW)], num_v)
    pltpu.sync_copy(wb_hbm, wb_v)

    pltpu.async_copy(tab_hbm.at[idx_v], val_v, sem).wait()

    lanes = jax.lax.iota(jnp.int32, LANES)
    for c in range(CHUNKS):
        sl = pl.ds(c * LANES, LANES)
        acc = wb_v[N_NUM, :]
        for i in range(N_NUM):
            acc = acc + num_v[i, sl] * wb_v[i, :]

        bv = (c * LANES + lanes) * HIST

        def hbody(h, a):
            return (a + plsc.load_gather(val_v, [bv + h])
                    + plsc.load_gather(val_v, [bv + (SEQN + h)]))

        acc = lax.fori_loop(0, HIST, hbody, acc)

        def tbody(t, a):
            return a + val_v[pl.ds(2 * SEQN + t * RPW + c * LANES, LANES)]

        acc = lax.fori_loop(0, N_CAT, tbody, acc)
        res_v[sl] = acc

    pltpu.sync_copy(res_v, out_hbm.at[pl.ds(base, RPW)])


@jax.jit
def _run(idx_w, num_all, wb, tab):
    mesh = plsc.VectorSubcoreMesh(core_axis_name="c", subcore_axis_name="s")
    fn = functools.partial(
        pl.kernel,
        mesh=mesh,
        compiler_params=pltpu.CompilerParams(needs_layout_passes=False,
                                             use_tc_tiling_on_sc=False),
        out_type=jax.ShapeDtypeStruct((B,), jnp.float32),
        scratch_types=[
            pltpu.VMEM((TOTN,), jnp.int32),
            pltpu.VMEM((TOTN,), jnp.float32),
            pltpu.VMEM((N_NUM, RPW), jnp.float32),
            pltpu.VMEM((N_NUM + 1, LANES), jnp.float32),
            pltpu.VMEM((RPW,), jnp.float32),
            pltpu.SemaphoreType.DMA,
        ],
    )(_sc_body)
    return fn(idx_w, num_all, wb, tab)


def kernel(seq_0, seq_0_table, seq_1, seq_1_table, cat_0, cat_0_table, cat_1, cat_1_table, cat_2, cat_2_table, cat_3, cat_3_table, cat_4, cat_4_table, cat_5, cat_5_table, cat_6, cat_6_table, cat_7, cat_7_table, cat_8, cat_8_table, cat_9, cat_9_table, cat_10, cat_10_table, cat_11, cat_11_table, cat_12, cat_12_table, cat_13, cat_13_table, cat_14, cat_14_table, cat_15, cat_15_table, cat_16, cat_16_table, cat_17, cat_17_table, cat_18, cat_18_table, cat_19, cat_19_table, cat_20, cat_20_table, cat_21, cat_21_table, cat_22, cat_22_table, cat_23, cat_23_table, cat_24, cat_24_table, cat_25, cat_25_table, num_0, num_1, num_2, num_3, num_4, num_5, num_6, num_7, num_8, num_9, num_10, num_11, num_12, dense_W, dense_b, bn_gamma, bn_beta, bn_mean, bn_var):
    cats = [cat_0, cat_1, cat_2, cat_3, cat_4, cat_5, cat_6, cat_7, cat_8,
            cat_9, cat_10, cat_11, cat_12, cat_13, cat_14, cat_15, cat_16,
            cat_17, cat_18, cat_19, cat_20, cat_21, cat_22, cat_23, cat_24,
            cat_25]
    cat_tables = [cat_0_table, cat_1_table, cat_2_table, cat_3_table,
                  cat_4_table, cat_5_table, cat_6_table, cat_7_table,
                  cat_8_table, cat_9_table, cat_10_table, cat_11_table,
                  cat_12_table, cat_13_table, cat_14_table, cat_15_table,
                  cat_16_table, cat_17_table, cat_18_table, cat_19_table,
                  cat_20_table, cat_21_table, cat_22_table, cat_23_table,
                  cat_24_table, cat_25_table]
    nums = [num_0, num_1, num_2, num_3, num_4, num_5, num_6, num_7, num_8,
            num_9, num_10, num_11, num_12]

    s0 = jnp.pad(seq_0_table, ((0, PAD_SEQ - 1000000), (0, 0))).reshape(-1)
    s1 = jnp.pad(seq_1_table, ((0, PAD_SEQ - 1000000), (0, 0))).reshape(-1)
    catp = jnp.pad(jnp.stack(cat_tables),
                   ((0, 0), (0, PAD_CAT - 100000), (0, 0)))
    tab = jnp.concatenate([s0, s1, catp.reshape(-1)])

    seq0_i = seq_0.astype(jnp.int32).reshape(NW, SEQN)
    seq1_i = (seq_1.astype(jnp.int32) + OFF_SEQ1).reshape(NW, SEQN)
    cat_off = OFF_CAT + PAD_CAT * jnp.arange(N_CAT, dtype=jnp.int32)
    cat_i = jnp.concatenate(
        [c.astype(jnp.int32).reshape(1, B) for c in cats],
        axis=0) + cat_off[:, None]
    cat_w = (cat_i.reshape(N_CAT, NW, RPW)
             .transpose(1, 0, 2).reshape(NW, CATN))
    idx_w = jnp.concatenate([seq0_i, seq1_i, cat_w], axis=1)

    num_all = jnp.stack(nums, axis=0).astype(jnp.float32)

    inv = bn_gamma / jnp.sqrt(bn_var + 1e-3)
    wfold = dense_W[:, 0] * inv
    bfold = dense_b[0] + jnp.sum((bn_beta - bn_mean * inv) * dense_W[:, 0])
    wb = jnp.broadcast_to(
        jnp.concatenate([wfold, bfold[None]]).astype(jnp.float32)[:, None],
        (N_NUM + 1, LANES))

    out = _run(idx_w, num_all, wb, tab)
    return out[:, None]


# speedup vs baseline: 3.8370x; 1.1025x over previous
import functools

import jax
import jax.numpy as jnp
from jax import lax
from jax.experimental import pallas as pl
from jax.experimental.pallas import tpu as pltpu
from jax.experimental.pallas import tpu_sc as plsc

B = 4096
HIST = 50
N_CAT = 26
N_NUM = 13
NC, NS = 2, 16
NW = NC * NS
RPW = B // NW
LANES = 16
CHUNKS = RPW // LANES
SEQN = HIST * RPW
CATN = N_CAT * RPW
TOTN = 2 * SEQN + CATN

PAD_CAT = 100352
PAD_SEQ = 1000448


def _sc_body(idx_hbm, num_hbm, wb_hbm, s0_hbm, s1_hbm, cat_hbm, out_hbm,
             idx_v, val_v, num_v, wb_v, res_v, sem0, sem1, sem2):
    wid = lax.axis_index("s") * NC + lax.axis_index("c")
    base = wid * RPW

    pltpu.sync_copy(idx_hbm.at[wid], idx_v)

    cp0 = pltpu.async_copy(s0_hbm.at[idx_v.at[pl.ds(0, SEQN)]],
                           val_v.at[pl.ds(0, SEQN)], sem0)
    cp1 = pltpu.async_copy(s1_hbm.at[idx_v.at[pl.ds(SEQN, SEQN)]],
                           val_v.at[pl.ds(SEQN, SEQN)], sem1)
    cp2 = pltpu.async_copy(cat_hbm.at[idx_v.at[pl.ds(2 * SEQN, CATN)]],
                           val_v.at[pl.ds(2 * SEQN, CATN)], sem2)

    pltpu.sync_copy(num_hbm.at[:, pl.ds(base, RPW)], num_v)
    pltpu.sync_copy(wb_hbm, wb_v)

    lanes = jax.lax.iota(jnp.int32, LANES)

    cp0.wait()
    for c in range(CHUNKS):
        sl = pl.ds(c * LANES, LANES)
        acc = wb_v[N_NUM, :]
        for i in range(N_NUM):
            acc = acc + num_v[i, sl] * wb_v[i, :]
        bv = (c * LANES + lanes) * HIST

        def h0(h, a):
            return a + plsc.load_gather(val_v, [bv + h])

        res_v[sl] = lax.fori_loop(0, HIST, h0, acc)

    cp1.wait()
    for c in range(CHUNKS):
        sl = pl.ds(c * LANES, LANES)
        bv = (c * LANES + lanes) * HIST + SEQN

        def h1(h, a):
            return a + plsc.load_gather(val_v, [bv + h])

        res_v[sl] = lax.fori_loop(0, HIST, h1, res_v[sl])

    cp2.wait()
    for c in range(CHUNKS):
        sl = pl.ds(c * LANES, LANES)

        def tb(t, a):
            return a + val_v[pl.ds(2 * SEQN + t * RPW + c * LANES, LANES)]

        res_v[sl] = lax.fori_loop(0, N_CAT, tb, res_v[sl])

    pltpu.sync_copy(res_v, out_hbm.at[pl.ds(base, RPW)])


@jax.jit
def _run(idx_w, num_all, wb, s0p, s1p, catf):
    mesh = plsc.VectorSubcoreMesh(core_axis_name="c", subcore_axis_name="s")
    fn = functools.partial(
        pl.kernel,
        mesh=mesh,
        compiler_params=pltpu.CompilerParams(needs_layout_passes=False,
                                             use_tc_tiling_on_sc=False),
        out_type=jax.ShapeDtypeStruct((B,), jnp.float32),
        scratch_types=[
            pltpu.VMEM((TOTN,), jnp.int32),
            pltpu.VMEM((TOTN,), jnp.float32),
            pltpu.VMEM((N_NUM, RPW), jnp.float32),
            pltpu.VMEM((N_NUM + 1, LANES), jnp.float32),
            pltpu.VMEM((RPW,), jnp.float32),
            pltpu.SemaphoreType.DMA,
            pltpu.SemaphoreType.DMA,
            pltpu.SemaphoreType.DMA,
        ],
    )(_sc_body)
    return fn(idx_w, num_all, wb, s0p, s1p, catf)


def kernel(seq_0, seq_0_table, seq_1, seq_1_table, cat_0, cat_0_table, cat_1, cat_1_table, cat_2, cat_2_table, cat_3, cat_3_table, cat_4, cat_4_table, cat_5, cat_5_table, cat_6, cat_6_table, cat_7, cat_7_table, cat_8, cat_8_table, cat_9, cat_9_table, cat_10, cat_10_table, cat_11, cat_11_table, cat_12, cat_12_table, cat_13, cat_13_table, cat_14, cat_14_table, cat_15, cat_15_table, cat_16, cat_16_table, cat_17, cat_17_table, cat_18, cat_18_table, cat_19, cat_19_table, cat_20, cat_20_table, cat_21, cat_21_table, cat_22, cat_22_table, cat_23, cat_23_table, cat_24, cat_24_table, cat_25, cat_25_table, num_0, num_1, num_2, num_3, num_4, num_5, num_6, num_7, num_8, num_9, num_10, num_11, num_12, dense_W, dense_b, bn_gamma, bn_beta, bn_mean, bn_var):
    cats = [cat_0, cat_1, cat_2, cat_3, cat_4, cat_5, cat_6, cat_7, cat_8,
            cat_9, cat_10, cat_11, cat_12, cat_13, cat_14, cat_15, cat_16,
            cat_17, cat_18, cat_19, cat_20, cat_21, cat_22, cat_23, cat_24,
            cat_25]
    cat_tables = [cat_0_table, cat_1_table, cat_2_table, cat_3_table,
                  cat_4_table, cat_5_table, cat_6_table, cat_7_table,
                  cat_8_table, cat_9_table, cat_10_table, cat_11_table,
                  cat_12_table, cat_13_table, cat_14_table, cat_15_table,
                  cat_16_table, cat_17_table, cat_18_table, cat_19_table,
                  cat_20_table, cat_21_table, cat_22_table, cat_23_table,
                  cat_24_table, cat_25_table]
    nums = [num_0, num_1, num_2, num_3, num_4, num_5, num_6, num_7, num_8,
            num_9, num_10, num_11, num_12]

    s0p = jnp.pad(seq_0_table, ((0, PAD_SEQ - 1000000), (0, 0))).reshape(-1)
    s1p = jnp.pad(seq_1_table, ((0, PAD_SEQ - 1000000), (0, 0))).reshape(-1)
    catf = jnp.pad(jnp.stack(cat_tables),
                   ((0, 0), (0, PAD_CAT - 100000), (0, 0))).reshape(-1)

    seq0_i = seq_0.astype(jnp.int32).reshape(NW, SEQN)
    seq1_i = seq_1.astype(jnp.int32).reshape(NW, SEQN)
    cat_off = PAD_CAT * jnp.arange(N_CAT, dtype=jnp.int32)
    cat_i = jnp.concatenate(
        [c.astype(jnp.int32).reshape(1, B) for c in cats],
        axis=0) + cat_off[:, None]
    cat_w = (cat_i.reshape(N_CAT, NW, RPW)
             .transpose(1, 0, 2).reshape(NW, CATN))
    idx_w = jnp.concatenate([seq0_i, seq1_i, cat_w], axis=1)

    num_all = jnp.stack(nums, axis=0).astype(jnp.float32)

    inv = bn_gamma / jnp.sqrt(bn_var + 1e-3)
    wfold = dense_W[:, 0] * inv
    bfold = dense_b[0] + jnp.sum((bn_beta - bn_mean * inv) * dense_W[:, 0])
    wb = jnp.broadcast_to(
        jnp.concatenate([wfold, bfold[None]]).astype(jnp.float32)[:, None],
        (N_NUM + 1, LANES))

    out = _run(idx_w, num_all, wb, s0p, s1p, catf)
    return out[:, None]


# speedup vs baseline: 4.2252x; 1.1012x over previous
import functools

import jax
import jax.numpy as jnp
from jax import lax
from jax.experimental import pallas as pl
from jax.experimental.pallas import tpu as pltpu
from jax.experimental.pallas import tpu_sc as plsc

B = 4096
HIST = 50
N_CAT = 26
N_NUM = 13
NC, NS = 2, 16
NW = NC * NS
RPW = B // NW
LANES = 16
CHUNKS = RPW // LANES
SEQN = HIST * RPW
CATN = N_CAT * RPW
TOTN = 2 * SEQN + CATN

PAD_CAT = 100352
PAD_SEQ = 1000448


def _sc_body(idx_hbm, num_hbm, wb_hbm, s0_hbm, s1_hbm, *rest):
    cat_hbms = rest[:N_CAT]
    out_hbm = rest[N_CAT]
    idx_v, val_v, num_v, wb_v, res_v, sem0, sem1, sem2 = rest[N_CAT + 1:]

    wid = lax.axis_index("s") * NC + lax.axis_index("c")
    base = wid * RPW

    pltpu.sync_copy(idx_hbm.at[wid], idx_v)

    cp0 = pltpu.async_copy(s0_hbm.at[idx_v.at[pl.ds(0, SEQN)]],
                           val_v.at[pl.ds(0, SEQN)], sem0)
    cp1 = pltpu.async_copy(s1_hbm.at[idx_v.at[pl.ds(SEQN, SEQN)]],
                           val_v.at[pl.ds(SEQN, SEQN)], sem1)
    cps = []
    for t in range(N_CAT):
        off = 2 * SEQN + t * RPW
        cps.append(pltpu.async_copy(cat_hbms[t].at[idx_v.at[pl.ds(off, RPW)]],
                                    val_v.at[pl.ds(off, RPW)], sem2))

    pltpu.sync_copy(num_hbm.at[:, pl.ds(base, RPW)], num_v)
    pltpu.sync_copy(wb_hbm, wb_v)

    lanes = jax.lax.iota(jnp.int32, LANES)

    cp0.wait()
    for c in range(CHUNKS):
        sl = pl.ds(c * LANES, LANES)
        acc = wb_v[N_NUM, :]
        for i in range(N_NUM):
            acc = acc + num_v[i, sl] * wb_v[i, :]
        bv = (c * LANES + lanes) * HIST

        def h0(h, a):
            return a + plsc.load_gather(val_v, [bv + h])

        res_v[sl] = lax.fori_loop(0, HIST, h0, acc)

    cp1.wait()
    for c in range(CHUNKS):
        sl = pl.ds(c * LANES, LANES)
        bv = (c * LANES + lanes) * HIST + SEQN

        def h1(h, a):
            return a + plsc.load_gather(val_v, [bv + h])

        res_v[sl] = lax.fori_loop(0, HIST, h1, res_v[sl])

    for cp in cps:
        cp.wait()
    for c in range(CHUNKS):
        sl = pl.ds(c * LANES, LANES)

        def tb(t, a):
            return a + val_v[pl.ds(2 * SEQN + t * RPW + c * LANES, LANES)]

        res_v[sl] = lax.fori_loop(0, N_CAT, tb, res_v[sl])

    pltpu.sync_copy(res_v, out_hbm.at[pl.ds(base, RPW)])


@jax.jit
def _run(idx_w, num_all, wb, s0p, s1p, *catps):
    mesh = plsc.VectorSubcoreMesh(core_axis_name="c", subcore_axis_name="s")
    fn = functools.partial(
        pl.kernel,
        mesh=mesh,
        compiler_params=pltpu.CompilerParams(needs_layout_passes=False,
                                             use_tc_tiling_on_sc=False),
        out_type=jax.ShapeDtypeStruct((B,), jnp.float32),
        scratch_types=[
            pltpu.VMEM((TOTN,), jnp.int32),
            pltpu.VMEM((TOTN,), jnp.float32),
            pltpu.VMEM((N_NUM, RPW), jnp.float32),
            pltpu.VMEM((N_NUM + 1, LANES), jnp.float32),
            pltpu.VMEM((RPW,), jnp.float32),
            pltpu.SemaphoreType.DMA,
            pltpu.SemaphoreType.DMA,
            pltpu.SemaphoreType.DMA,
        ],
    )(_sc_body)
    return fn(idx_w, num_all, wb, s0p, s1p, *catps)


def kernel(seq_0, seq_0_table, seq_1, seq_1_table, cat_0, cat_0_table, cat_1, cat_1_table, cat_2, cat_2_table, cat_3, cat_3_table, cat_4, cat_4_table, cat_5, cat_5_table, cat_6, cat_6_table, cat_7, cat_7_table, cat_8, cat_8_table, cat_9, cat_9_table, cat_10, cat_10_table, cat_11, cat_11_table, cat_12, cat_12_table, cat_13, cat_13_table, cat_14, cat_14_table, cat_15, cat_15_table, cat_16, cat_16_table, cat_17, cat_17_table, cat_18, cat_18_table, cat_19, cat_19_table, cat_20, cat_20_table, cat_21, cat_21_table, cat_22, cat_22_table, cat_23, cat_23_table, cat_24, cat_24_table, cat_25, cat_25_table, num_0, num_1, num_2, num_3, num_4, num_5, num_6, num_7, num_8, num_9, num_10, num_11, num_12, dense_W, dense_b, bn_gamma, bn_beta, bn_mean, bn_var):
    cats = [cat_0, cat_1, cat_2, cat_3, cat_4, cat_5, cat_6, cat_7, cat_8,
            cat_9, cat_10, cat_11, cat_12, cat_13, cat_14, cat_15, cat_16,
            cat_17, cat_18, cat_19, cat_20, cat_21, cat_22, cat_23, cat_24,
            cat_25]
    cat_tables = [cat_0_table, cat_1_table, cat_2_table, cat_3_table,
                  cat_4_table, cat_5_table, cat_6_table, cat_7_table,
                  cat_8_table, cat_9_table, cat_10_table, cat_11_table,
                  cat_12_table, cat_13_table, cat_14_table, cat_15_table,
                  cat_16_table, cat_17_table, cat_18_table, cat_19_table,
                  cat_20_table, cat_21_table, cat_22_table, cat_23_table,
                  cat_24_table, cat_25_table]
    nums = [num_0, num_1, num_2, num_3, num_4, num_5, num_6, num_7, num_8,
            num_9, num_10, num_11, num_12]

    s0p = jnp.pad(seq_0_table, ((0, PAD_SEQ - 1000000), (0, 0))).reshape(-1)
    s1p = jnp.pad(seq_1_table, ((0, PAD_SEQ - 1000000), (0, 0))).reshape(-1)
    catps = [jnp.pad(t, ((0, PAD_CAT - 100000), (0, 0))).reshape(-1)
             for t in cat_tables]

    seq0_i = seq_0.astype(jnp.int32).reshape(NW, SEQN)
    seq1_i = seq_1.astype(jnp.int32).reshape(NW, SEQN)
    cat_i = jnp.concatenate(
        [c.astype(jnp.int32).reshape(1, B) for c in cats], axis=0)
    cat_w = (cat_i.reshape(N_CAT, NW, RPW)
             .transpose(1, 0, 2).reshape(NW, CATN))
    idx_w = jnp.concatenate([seq0_i, seq1_i, cat_w], axis=1)

    num_all = jnp.stack(nums, axis=0).astype(jnp.float32)

    inv = bn_gamma / jnp.sqrt(bn_var + 1e-3)
    wfold = dense_W[:, 0] * inv
    bfold = dense_b[0] + jnp.sum((bn_beta - bn_mean * inv) * dense_W[:, 0])
    wb = jnp.broadcast_to(
        jnp.concatenate([wfold, bfold[None]]).astype(jnp.float32)[:, None],
        (N_NUM + 1, LANES))

    out = _run(idx_w, num_all, wb, s0p, s1p, *catps)
    return out[:, None]
